# baseline (speedup 1.0000x reference)
import jax
import jax.numpy as jnp
from jax.experimental import pallas as pl
from jax.experimental.pallas import tpu as pltpu
from jax.experimental.pallas import tpu_sc as plsc

B = 2048
D = 768
E = 8
H = 3072
TILE = 512
NA = 2 * B
NT = NA // TILE + E
PT = NT * TILE
SCW = 64


def _meta_kernel(x_ref, gw_ref, gb_ref, gates_ref, p_ref, te_ref, tn_ref):
    logits = jnp.dot(x_ref[...], gw_ref[...],
                     preferred_element_type=jnp.float32) + gb_ref[0]
    iota = jax.lax.broadcasted_iota(jnp.int32, (B, E), 1)
    m1 = jnp.max(logits, axis=1, keepdims=True)
    idx1 = jnp.min(jnp.where(logits == m1, iota, E), axis=1, keepdims=True)
    masked = jnp.where(iota == idx1, -jnp.inf, logits)
    m2 = jnp.max(masked, axis=1, keepdims=True)
    idx2 = jnp.min(jnp.where((masked == m2) & (iota != idx1), iota, E),
                   axis=1, keepdims=True)
    g1 = 1.0 / (1.0 + jnp.exp(m2 - m1))
    gates_ref[...] = jnp.concatenate([g1, 1.0 - g1], axis=1)

    oh0 = (iota == idx1).astype(jnp.float32)
    oh1 = (iota == idx2).astype(jnp.float32)

    counts0 = jnp.sum(oh0, axis=0, keepdims=True)
    counts = counts0 + jnp.sum(oh1, axis=0, keepdims=True)
    counts_i = counts.astype(jnp.int32)
    pad_i = ((counts_i + (TILE - 1)) // TILE) * TILE
    i8r = jax.lax.broadcasted_iota(jnp.int32, (E, E), 0)
    i8c = jax.lax.broadcasted_iota(jnp.int32, (E, E), 1)
    s8 = (i8r < i8c).astype(jnp.bfloat16)
    offs = jnp.dot(pad_i.astype(jnp.bfloat16), s8,
                   preferred_element_type=jnp.float32)

    nblk = B // TILE
    ir = jax.lax.broadcasted_iota(jnp.int32, (TILE, TILE), 0)
    ic = jax.lax.broadcasted_iota(jnp.int32, (TILE, TILE), 1)
    ltri = (ir > ic).astype(jnp.bfloat16)

    def rank_pass(oh, k, carry):
        for j in range(nblk):
            ohb = oh[j * TILE:(j + 1) * TILE, :]
            cum = jnp.dot(ltri, ohb.astype(jnp.bfloat16),
                          preferred_element_type=jnp.float32) + carry
            rank = jnp.sum(cum * ohb, axis=1, keepdims=True)
            base = jnp.sum(offs * ohb, axis=1, keepdims=True)
            p_ref[j * TILE:(j + 1) * TILE, k:k + 1] = (
                (base + rank).astype(jnp.int32))
            carry = carry + jnp.sum(ohb, axis=0, keepdims=True)
        return carry

    carry0 = rank_pass(oh0, 0, jnp.zeros((1, E), jnp.float32))
    rank_pass(oh1, 1, carry0)

    tstart = (jax.lax.broadcasted_iota(jnp.int32, (NT, 1), 0)
              .astype(jnp.float32) * TILE)
    offs_b = jnp.broadcast_to(offs, (NT, E))
    te = jnp.sum((offs_b <= tstart).astype(jnp.float32), axis=1,
                 keepdims=True) - 1.0
    i8t = jax.lax.broadcasted_iota(jnp.int32, (NT, E), 1).astype(jnp.float32)
    ohte = (i8t == te).astype(jnp.float32)
    offs_te = jnp.sum(offs_b * ohte, axis=1, keepdims=True)
    counts_te = jnp.sum(jnp.broadcast_to(counts, (NT, E)) * ohte,
                        axis=1, keepdims=True)
    nrows = jnp.clip(counts_te - (tstart - offs_te), 0.0, float(TILE))
    te_ref[...] = te.astype(jnp.int32)
    tn_ref[...] = nrows.astype(jnp.int32)


HB = 1536
NPASS = H // HB


def _gffn_kernel(te_ref, tn_ref, xs_ref, w1_ref, b1_ref, w2_ref, b2_ref,
                 ys_ref, w1bf_ref, w2bf_ref):
    hb = pl.program_id(0)
    t = pl.program_id(1)
    live = tn_ref[t] > 0
    changed = jnp.logical_or(t == 0,
                             te_ref[t] != te_ref[jnp.maximum(t - 1, 0)])

    @pl.when(jnp.logical_and(live, changed))
    def _():
        w1bf_ref[...] = w1_ref[0].astype(jnp.bfloat16)
        w2bf_ref[...] = w2_ref[0].astype(jnp.bfloat16)

    @pl.when(live)
    def _():
        xb = xs_ref[...].astype(jnp.bfloat16)
        h = jnp.dot(xb, w1bf_ref[...], preferred_element_type=jnp.float32)
        h = jnp.maximum(h + b1_ref[0, 0], 0.0).astype(jnp.bfloat16)
        y = jnp.dot(h, w2bf_ref[...], preferred_element_type=jnp.float32)
        sl = pl.ds(t * TILE, TILE)

        @pl.when(hb == 0)
        def _():
            ys_ref[sl, :] = y + b2_ref[0, 0]

        @pl.when(hb != 0)
        def _():
            ys_ref[sl, :] += y


def _combine_kernel(g_ref, y_ref, out_ref):
    out_ref[...] = (g_ref[:, 0:1] * y_ref[0] + g_ref[:, 1:2] * y_ref[1])


def kernel(x, gate_w, gate_b, w1, b1, w2, b2):
    gates, p, te, tn = pl.pallas_call(
        _meta_kernel,
        out_shape=(
            jax.ShapeDtypeStruct((B, 2), jnp.float32),
            jax.ShapeDtypeStruct((B, 2), jnp.int32),
            jax.ShapeDtypeStruct((NT, 1), jnp.int32),
            jax.ShapeDtypeStruct((NT, 1), jnp.int32),
        ),
        in_specs=[
            pl.BlockSpec((B, D), lambda: (0, 0)),
            pl.BlockSpec((D, E), lambda: (0, 0)),
            pl.BlockSpec((1, E), lambda: (0, 0)),
        ],
        out_specs=(
            pl.BlockSpec((B, 2), lambda: (0, 0)),
            pl.BlockSpec((B, 2), lambda: (0, 0)),
            pl.BlockSpec((NT, 1), lambda: (0, 0)),
            pl.BlockSpec((NT, 1), lambda: (0, 0)),
        ),
    )(x, gate_w, gate_b.reshape(1, E))

    p_t = p.T.reshape(2, B)
    te_flat = te.reshape(NT)
    tn_flat = tn.reshape(NT)

    vmesh = plsc.VectorSubcoreMesh(core_axis_name="c", subcore_axis_name="s")
    NSUB = 32
    CH = NA // NSUB

    def sc_scatter(xv, idx):
        @pl.kernel(out_type=jax.ShapeDtypeStruct((PT, D), jnp.float32),
                   mesh=vmesh,
                   scratch_types=[pltpu.VMEM((1, CH), jnp.int32),
                                  pltpu.VMEM((CH, D), jnp.float32),
                                  pltpu.SemaphoreType.DMA,
                                  pltpu.SemaphoreType.DMA])
        def _scatter(x_hbm, i_hbm, o_hbm, ibuf, xbuf, sem1, sem2):
            c = jax.lax.axis_index("c")
            s = jax.lax.axis_index("s")
            sub = c * 16 + s
            k = sub // 16
            b0 = (sub % 16) * CH
            cp1 = pltpu.async_copy(i_hbm.at[pl.ds(k, 1), pl.ds(b0, CH)],
                                   ibuf, sem1)
            cp2 = pltpu.async_copy(x_hbm.at[pl.ds(b0, CH), :], xbuf, sem2)
            cp1.wait()
            cp2.wait()
            pltpu.sync_copy(xbuf, o_hbm.at[ibuf.at[0]])
        return _scatter(xv, idx)

    xs = sc_scatter(x, p_t)

    ys = pl.pallas_call(
        _gffn_kernel,
        grid_spec=pltpu.PrefetchScalarGridSpec(
            num_scalar_prefetch=2,
            grid=(NPASS, NT),
            in_specs=[
                pl.BlockSpec((TILE, D), lambda hb, t, te_, tn_: (t, 0)),
                pl.BlockSpec((1, D, HB),
                             lambda hb, t, te_, tn_: (te_[t], 0, hb)),
                pl.BlockSpec((1, 1, HB),
                             lambda hb, t, te_, tn_: (te_[t], 0, hb)),
                pl.BlockSpec((1, HB, D),
                             lambda hb, t, te_, tn_: (te_[t], hb, 0)),
                pl.BlockSpec((1, 1, D),
                             lambda hb, t, te_, tn_: (te_[t], 0, 0)),
            ],
            out_specs=pl.BlockSpec((PT, D), lambda hb, t, te_, tn_: (0, 0)),
            scratch_shapes=[
                pltpu.VMEM((D, HB), jnp.bfloat16),
                pltpu.VMEM((HB, D), jnp.bfloat16),
            ],
        ),
        out_shape=jax.ShapeDtypeStruct((PT, D), jnp.float32),
        compiler_params=pltpu.CompilerParams(
            dimension_semantics=("arbitrary", "arbitrary"),
        ),
    )(te_flat, tn_flat, xs, w1, b1.reshape(E, 1, H), w2,
      b2.reshape(E, 1, D))

    def sc_gather(yv, idx):
        @pl.kernel(out_type=jax.ShapeDtypeStruct((NA, D), jnp.float32),
                   mesh=vmesh,
                   scratch_types=[pltpu.VMEM((1, CH), jnp.int32),
                                  pltpu.VMEM((CH, D), jnp.float32),
                                  pltpu.SemaphoreType.DMA])
        def _gather(y_hbm, i_hbm, o_hbm, ibuf, ybuf, sem1):
            c = jax.lax.axis_index("c")
            s = jax.lax.axis_index("s")
            sub = c * 16 + s
            a0 = sub * CH
            pltpu.async_copy(i_hbm.at[pl.ds(0, 1), pl.ds(a0, CH)],
                             ibuf, sem1).wait()
            pltpu.sync_copy(y_hbm.at[ibuf.at[0]], ybuf)
            pltpu.async_copy(ybuf, o_hbm.at[pl.ds(a0, CH), :], sem1).wait()
        return _gather(yv, idx)

    yg = sc_gather(ys, p_t.reshape(1, NA)).reshape(2, B, D)

    out = pl.pallas_call(
        _combine_kernel,
        out_shape=jax.ShapeDtypeStruct((B, D), jnp.float32),
        in_specs=[
            pl.BlockSpec((B, 2), lambda: (0, 0)),
            pl.BlockSpec((2, B, D), lambda: (0, 0, 0)),
        ],
        out_specs=pl.BlockSpec((B, D), lambda: (0, 0)),
    )(gates, yg)
    return out


# speedup vs baseline: 1.1154x; 1.1154x over previous
import jax
import jax.numpy as jnp
from jax.experimental import pallas as pl
from jax.experimental.pallas import tpu as pltpu
from jax.experimental.pallas import tpu_sc as plsc

B = 2048
D = 768
E = 8
H = 3072
TILE = 512
NA = 2 * B
NT = NA // TILE + E
PT = NT * TILE
SCW = 64


def _meta_kernel(x_ref, gw_ref, gb_ref, gates_ref, p_ref, te_ref, tn_ref):
    logits = jnp.dot(x_ref[...], gw_ref[...],
                     preferred_element_type=jnp.float32) + gb_ref[0]
    iota = jax.lax.broadcasted_iota(jnp.int32, (B, E), 1)
    m1 = jnp.max(logits, axis=1, keepdims=True)
    idx1 = jnp.min(jnp.where(logits == m1, iota, E), axis=1, keepdims=True)
    masked = jnp.where(iota == idx1, -jnp.inf, logits)
    m2 = jnp.max(masked, axis=1, keepdims=True)
    idx2 = jnp.min(jnp.where((masked == m2) & (iota != idx1), iota, E),
                   axis=1, keepdims=True)
    g1 = 1.0 / (1.0 + jnp.exp(m2 - m1))
    gates_ref[...] = jnp.concatenate([g1, 1.0 - g1], axis=1)

    oh0 = (iota == idx1).astype(jnp.float32)
    oh1 = (iota == idx2).astype(jnp.float32)

    counts0 = jnp.sum(oh0, axis=0, keepdims=True)
    counts = counts0 + jnp.sum(oh1, axis=0, keepdims=True)
    counts_i = counts.astype(jnp.int32)
    pad_i = ((counts_i + (TILE - 1)) // TILE) * TILE
    i8r = jax.lax.broadcasted_iota(jnp.int32, (E, E), 0)
    i8c = jax.lax.broadcasted_iota(jnp.int32, (E, E), 1)
    s8 = (i8r < i8c).astype(jnp.bfloat16)
    offs = jnp.dot(pad_i.astype(jnp.bfloat16), s8,
                   preferred_element_type=jnp.float32)

    nblk = B // TILE
    ir = jax.lax.broadcasted_iota(jnp.int32, (TILE, TILE), 0)
    ic = jax.lax.broadcasted_iota(jnp.int32, (TILE, TILE), 1)
    ltri = (ir > ic).astype(jnp.bfloat16)

    def rank_pass(oh, k, carry):
        for j in range(nblk):
            ohb = oh[j * TILE:(j + 1) * TILE, :]
            cum = jnp.dot(ltri, ohb.astype(jnp.bfloat16),
                          preferred_element_type=jnp.float32) + carry
            rank = jnp.sum(cum * ohb, axis=1, keepdims=True)
            base = jnp.sum(offs * ohb, axis=1, keepdims=True)
            p_ref[j * TILE:(j + 1) * TILE, k:k + 1] = (
                (base + rank).astype(jnp.int32))
            carry = carry + jnp.sum(ohb, axis=0, keepdims=True)
        return carry

    carry0 = rank_pass(oh0, 0, jnp.zeros((1, E), jnp.float32))
    rank_pass(oh1, 1, carry0)

    tstart = (jax.lax.broadcasted_iota(jnp.int32, (NT, 1), 0)
              .astype(jnp.float32) * TILE)
    offs_b = jnp.broadcast_to(offs, (NT, E))
    te = jnp.sum((offs_b <= tstart).astype(jnp.float32), axis=1,
                 keepdims=True) - 1.0
    i8t = jax.lax.broadcasted_iota(jnp.int32, (NT, E), 1).astype(jnp.float32)
    ohte = (i8t == te).astype(jnp.float32)
    offs_te = jnp.sum(offs_b * ohte, axis=1, keepdims=True)
    counts_te = jnp.sum(jnp.broadcast_to(counts, (NT, E)) * ohte,
                        axis=1, keepdims=True)
    nrows = jnp.clip(counts_te - (tstart - offs_te), 0.0, float(TILE))
    te_ref[...] = te.astype(jnp.int32)
    tn_ref[...] = nrows.astype(jnp.int32)


def _gffn_kernel(te_ref, tn_ref, xs_ref, w1_ref, b1_ref, w2_ref, b2_ref,
                 ys_ref, w1bf_ref, w2bf_ref):
    t = pl.program_id(0)
    live = tn_ref[t] > 0
    changed = jnp.logical_or(t == 0,
                             te_ref[t] != te_ref[jnp.maximum(t - 1, 0)])

    @pl.when(jnp.logical_and(live, changed))
    def _():
        w1bf_ref[...] = w1_ref[0].astype(jnp.bfloat16)
        w2bf_ref[...] = w2_ref[0].astype(jnp.bfloat16)

    @pl.when(live)
    def _():
        xb = xs_ref[...].astype(jnp.bfloat16)
        h = jnp.dot(xb, w1bf_ref[...], preferred_element_type=jnp.float32)
        h = jnp.maximum(h + b1_ref[0, 0], 0.0).astype(jnp.bfloat16)
        y = jnp.dot(h, w2bf_ref[...], preferred_element_type=jnp.float32)
        ys_ref[...] = y + b2_ref[0, 0]


def _combine_kernel(g_ref, y_ref, out_ref):
    out_ref[...] = (g_ref[:, 0:1] * y_ref[0] + g_ref[:, 1:2] * y_ref[1])


def kernel(x, gate_w, gate_b, w1, b1, w2, b2):
    gates, p, te, tn = pl.pallas_call(
        _meta_kernel,
        out_shape=(
            jax.ShapeDtypeStruct((B, 2), jnp.float32),
            jax.ShapeDtypeStruct((B, 2), jnp.int32),
            jax.ShapeDtypeStruct((NT, 1), jnp.int32),
            jax.ShapeDtypeStruct((NT, 1), jnp.int32),
        ),
        in_specs=[
            pl.BlockSpec((B, D), lambda: (0, 0)),
            pl.BlockSpec((D, E), lambda: (0, 0)),
            pl.BlockSpec((1, E), lambda: (0, 0)),
        ],
        out_specs=(
            pl.BlockSpec((B, 2), lambda: (0, 0)),
            pl.BlockSpec((B, 2), lambda: (0, 0)),
            pl.BlockSpec((NT, 1), lambda: (0, 0)),
            pl.BlockSpec((NT, 1), lambda: (0, 0)),
        ),
    )(x, gate_w, gate_b.reshape(1, E))

    p_t = p.T.reshape(2, B)
    te_flat = te.reshape(NT)
    tn_flat = tn.reshape(NT)

    vmesh = plsc.VectorSubcoreMesh(core_axis_name="c", subcore_axis_name="s")
    NSUB = 32
    CH = NA // NSUB

    def sc_scatter(xv, idx):
        @pl.kernel(out_type=jax.ShapeDtypeStruct((PT, D), jnp.float32),
                   mesh=vmesh,
                   scratch_types=[pltpu.VMEM((1, CH), jnp.int32),
                                  pltpu.VMEM((CH, D), jnp.float32),
                                  pltpu.SemaphoreType.DMA,
                                  pltpu.SemaphoreType.DMA])
        def _scatter(x_hbm, i_hbm, o_hbm, ibuf, xbuf, sem1, sem2):
            c = jax.lax.axis_index("c")
            s = jax.lax.axis_index("s")
            sub = c * 16 + s
            k = sub // 16
            b0 = (sub % 16) * CH
            cp1 = pltpu.async_copy(i_hbm.at[pl.ds(k, 1), pl.ds(b0, CH)],
                                   ibuf, sem1)
            cp2 = pltpu.async_copy(x_hbm.at[pl.ds(b0, CH), :], xbuf, sem2)
            cp1.wait()
            cp2.wait()
            pltpu.sync_copy(xbuf, o_hbm.at[ibuf.at[0]])
        return _scatter(xv, idx)

    xs = sc_scatter(x, p_t)

    ys = pl.pallas_call(
        _gffn_kernel,
        grid_spec=pltpu.PrefetchScalarGridSpec(
            num_scalar_prefetch=2,
            grid=(NT,),
            in_specs=[
                pl.BlockSpec((TILE, D), lambda t, te_, tn_: (t, 0)),
                pl.BlockSpec((1, D, H), lambda t, te_, tn_: (te_[t], 0, 0)),
                pl.BlockSpec((1, 1, H), lambda t, te_, tn_: (te_[t], 0, 0)),
                pl.BlockSpec((1, H, D), lambda t, te_, tn_: (te_[t], 0, 0)),
                pl.BlockSpec((1, 1, D), lambda t, te_, tn_: (te_[t], 0, 0)),
            ],
            out_specs=pl.BlockSpec((TILE, D), lambda t, te_, tn_: (t, 0)),
            scratch_shapes=[
                pltpu.VMEM((D, H), jnp.bfloat16),
                pltpu.VMEM((H, D), jnp.bfloat16),
            ],
        ),
        out_shape=jax.ShapeDtypeStruct((PT, D), jnp.float32),
        compiler_params=pltpu.CompilerParams(
            dimension_semantics=("arbitrary",),
        ),
    )(te_flat, tn_flat, xs, w1, b1.reshape(E, 1, H), w2,
      b2.reshape(E, 1, D))

    def sc_gather(yv, idx):
        @pl.kernel(out_type=jax.ShapeDtypeStruct((NA, D), jnp.float32),
                   mesh=vmesh,
                   scratch_types=[pltpu.VMEM((1, CH), jnp.int32),
                                  pltpu.VMEM((CH, D), jnp.float32),
                                  pltpu.SemaphoreType.DMA])
        def _gather(y_hbm, i_hbm, o_hbm, ibuf, ybuf, sem1):
            c = jax.lax.axis_index("c")
            s = jax.lax.axis_index("s")
            sub = c * 16 + s
            a0 = sub * CH
            pltpu.async_copy(i_hbm.at[pl.ds(0, 1), pl.ds(a0, CH)],
                             ibuf, sem1).wait()
            pltpu.sync_copy(y_hbm.at[ibuf.at[0]], ybuf)
            pltpu.async_copy(ybuf, o_hbm.at[pl.ds(a0, CH), :], sem1).wait()
        return _gather(yv, idx)

    yg = sc_gather(ys, p_t.reshape(1, NA)).reshape(2, B, D)

    out = pl.pallas_call(
        _combine_kernel,
        out_shape=jax.ShapeDtypeStruct((B, D), jnp.float32),
        in_specs=[
            pl.BlockSpec((B, 2), lambda: (0, 0)),
            pl.BlockSpec((2, B, D), lambda: (0, 0, 0)),
        ],
        out_specs=pl.BlockSpec((B, D), lambda: (0, 0)),
    )(gates, yg)
    return out


# speedup vs baseline: 1.1573x; 1.0375x over previous
import jax
import jax.numpy as jnp
from jax.experimental import pallas as pl
from jax.experimental.pallas import tpu as pltpu
from jax.experimental.pallas import tpu_sc as plsc

B = 2048
D = 768
E = 8
H = 3072
TILE = 512
NA = 2 * B
NT = NA // TILE + E
PT = NT * TILE
SCW = 64


def _meta_kernel(x_ref, gw_ref, gb_ref, gates_ref, p_ref, te_ref, tn_ref):
    logits = jnp.dot(x_ref[...], gw_ref[...],
                     preferred_element_type=jnp.float32) + gb_ref[0]
    iota = jax.lax.broadcasted_iota(jnp.int32, (B, E), 1)
    m1 = jnp.max(logits, axis=1, keepdims=True)
    idx1 = jnp.min(jnp.where(logits == m1, iota, E), axis=1, keepdims=True)
    masked = jnp.where(iota == idx1, -jnp.inf, logits)
    m2 = jnp.max(masked, axis=1, keepdims=True)
    idx2 = jnp.min(jnp.where((masked == m2) & (iota != idx1), iota, E),
                   axis=1, keepdims=True)
    g1 = 1.0 / (1.0 + jnp.exp(m2 - m1))
    gates_ref[...] = jnp.concatenate([g1, 1.0 - g1], axis=1)

    oh0 = (iota == idx1).astype(jnp.float32)
    oh1 = (iota == idx2).astype(jnp.float32)

    counts0 = jnp.sum(oh0, axis=0, keepdims=True)
    counts = counts0 + jnp.sum(oh1, axis=0, keepdims=True)
    counts_i = counts.astype(jnp.int32)
    pad_i = ((counts_i + (TILE - 1)) // TILE) * TILE
    i8r = jax.lax.broadcasted_iota(jnp.int32, (E, E), 0)
    i8c = jax.lax.broadcasted_iota(jnp.int32, (E, E), 1)
    s8 = (i8r < i8c).astype(jnp.bfloat16)
    offs = jnp.dot(pad_i.astype(jnp.bfloat16), s8,
                   preferred_element_type=jnp.float32)

    nblk = B // TILE
    ir = jax.lax.broadcasted_iota(jnp.int32, (TILE, TILE), 0)
    ic = jax.lax.broadcasted_iota(jnp.int32, (TILE, TILE), 1)
    ltri = (ir > ic).astype(jnp.bfloat16)

    def rank_pass(oh, k, carry):
        for j in range(nblk):
            ohb = oh[j * TILE:(j + 1) * TILE, :]
            cum = jnp.dot(ltri, ohb.astype(jnp.bfloat16),
                          preferred_element_type=jnp.float32) + carry
            rank = jnp.sum(cum * ohb, axis=1, keepdims=True)
            base = jnp.sum(offs * ohb, axis=1, keepdims=True)
            p_ref[j * TILE:(j + 1) * TILE, k:k + 1] = (
                (base + rank).astype(jnp.int32))
            carry = carry + jnp.sum(ohb, axis=0, keepdims=True)
        return carry

    carry0 = rank_pass(oh0, 0, jnp.zeros((1, E), jnp.float32))
    rank_pass(oh1, 1, carry0)

    tstart = (jax.lax.broadcasted_iota(jnp.int32, (NT, 1), 0)
              .astype(jnp.float32) * TILE)
    offs_b = jnp.broadcast_to(offs, (NT, E))
    te = jnp.sum((offs_b <= tstart).astype(jnp.float32), axis=1,
                 keepdims=True) - 1.0
    i8t = jax.lax.broadcasted_iota(jnp.int32, (NT, E), 1).astype(jnp.float32)
    ohte = (i8t == te).astype(jnp.float32)
    offs_te = jnp.sum(offs_b * ohte, axis=1, keepdims=True)
    counts_te = jnp.sum(jnp.broadcast_to(counts, (NT, E)) * ohte,
                        axis=1, keepdims=True)
    nrows = jnp.clip(counts_te - (tstart - offs_te), 0.0, float(TILE))
    te_ref[...] = te.astype(jnp.int32)
    tn_ref[...] = nrows.astype(jnp.int32)


def _gffn_kernel(te_ref, tn_ref, xs_ref, w1_ref, b1_ref, w2_ref, b2_ref,
                 ys_ref):
    t = pl.program_id(0)

    @pl.when(tn_ref[t] > 0)
    def _():
        xb = xs_ref[...].astype(jnp.bfloat16)
        h = jnp.dot(xb, w1_ref[0].astype(jnp.bfloat16),
                    preferred_element_type=jnp.float32)
        h = jnp.maximum(h + b1_ref[0, 0], 0.0).astype(jnp.bfloat16)
        y = jnp.dot(h, w2_ref[0].astype(jnp.bfloat16),
                    preferred_element_type=jnp.float32)
        ys_ref[...] = y + b2_ref[0, 0]


def _combine_kernel(g_ref, y_ref, out_ref):
    out_ref[...] = (g_ref[:, 0:1] * y_ref[0] + g_ref[:, 1:2] * y_ref[1])


def kernel(x, gate_w, gate_b, w1, b1, w2, b2):
    gates, p, te, tn = pl.pallas_call(
        _meta_kernel,
        out_shape=(
            jax.ShapeDtypeStruct((B, 2), jnp.float32),
            jax.ShapeDtypeStruct((B, 2), jnp.int32),
            jax.ShapeDtypeStruct((NT, 1), jnp.int32),
            jax.ShapeDtypeStruct((NT, 1), jnp.int32),
        ),
        in_specs=[
            pl.BlockSpec((B, D), lambda: (0, 0)),
            pl.BlockSpec((D, E), lambda: (0, 0)),
            pl.BlockSpec((1, E), lambda: (0, 0)),
        ],
        out_specs=(
            pl.BlockSpec((B, 2), lambda: (0, 0)),
            pl.BlockSpec((B, 2), lambda: (0, 0)),
            pl.BlockSpec((NT, 1), lambda: (0, 0)),
            pl.BlockSpec((NT, 1), lambda: (0, 0)),
        ),
    )(x, gate_w, gate_b.reshape(1, E))

    p_t = p.T.reshape(2, B)
    te_flat = te.reshape(NT)
    tn_flat = tn.reshape(NT)

    vmesh = plsc.VectorSubcoreMesh(core_axis_name="c", subcore_axis_name="s")
    NSUB = 32
    CH = NA // NSUB

    def sc_scatter(xv, idx):
        @pl.kernel(out_type=jax.ShapeDtypeStruct((PT, D), jnp.float32),
                   mesh=vmesh,
                   scratch_types=[pltpu.VMEM((1, CH), jnp.int32),
                                  pltpu.VMEM((CH, D), jnp.float32),
                                  pltpu.SemaphoreType.DMA,
                                  pltpu.SemaphoreType.DMA])
        def _scatter(x_hbm, i_hbm, o_hbm, ibuf, xbuf, sem1, sem2):
            c = jax.lax.axis_index("c")
            s = jax.lax.axis_index("s")
            sub = c * 16 + s
            k = sub // 16
            b0 = (sub % 16) * CH
            cp1 = pltpu.async_copy(i_hbm.at[pl.ds(k, 1), pl.ds(b0, CH)],
                                   ibuf, sem1)
            cp2 = pltpu.async_copy(x_hbm.at[pl.ds(b0, CH), :], xbuf, sem2)
            cp1.wait()
            cp2.wait()
            pltpu.sync_copy(xbuf, o_hbm.at[ibuf.at[0]])
        return _scatter(xv, idx)

    xs = sc_scatter(x, p_t)

    ys = pl.pallas_call(
        _gffn_kernel,
        grid_spec=pltpu.PrefetchScalarGridSpec(
            num_scalar_prefetch=2,
            grid=(NT,),
            in_specs=[
                pl.BlockSpec((TILE, D), lambda t, te_, tn_: (t, 0)),
                pl.BlockSpec((1, D, H), lambda t, te_, tn_: (te_[t], 0, 0)),
                pl.BlockSpec((1, 1, H), lambda t, te_, tn_: (te_[t], 0, 0)),
                pl.BlockSpec((1, H, D), lambda t, te_, tn_: (te_[t], 0, 0)),
                pl.BlockSpec((1, 1, D), lambda t, te_, tn_: (te_[t], 0, 0)),
            ],
            out_specs=pl.BlockSpec((TILE, D), lambda t, te_, tn_: (t, 0)),
        ),
        out_shape=jax.ShapeDtypeStruct((PT, D), jnp.float32),
        compiler_params=pltpu.CompilerParams(
            dimension_semantics=("arbitrary",),
        ),
    )(te_flat, tn_flat, xs, w1, b1.reshape(E, 1, H), w2,
      b2.reshape(E, 1, D))

    def sc_gather(yv, idx):
        @pl.kernel(out_type=jax.ShapeDtypeStruct((NA, D), jnp.float32),
                   mesh=vmesh,
                   scratch_types=[pltpu.VMEM((1, CH), jnp.int32),
                                  pltpu.VMEM((CH, D), jnp.float32),
                                  pltpu.SemaphoreType.DMA])
        def _gather(y_hbm, i_hbm, o_hbm, ibuf, ybuf, sem1):
            c = jax.lax.axis_index("c")
            s = jax.lax.axis_index("s")
            sub = c * 16 + s
            a0 = sub * CH
            pltpu.async_copy(i_hbm.at[pl.ds(0, 1), pl.ds(a0, CH)],
                             ibuf, sem1).wait()
            pltpu.sync_copy(y_hbm.at[ibuf.at[0]], ybuf)
            pltpu.async_copy(ybuf, o_hbm.at[pl.ds(a0, CH), :], sem1).wait()
        return _gather(yv, idx)

    yg = sc_gather(ys, p_t.reshape(1, NA)).reshape(2, B, D)

    out = pl.pallas_call(
        _combine_kernel,
        out_shape=jax.ShapeDtypeStruct((B, D), jnp.float32),
        in_specs=[
            pl.BlockSpec((B, 2), lambda: (0, 0)),
            pl.BlockSpec((2, B, D), lambda: (0, 0, 0)),
        ],
        out_specs=pl.BlockSpec((B, D), lambda: (0, 0)),
    )(gates, yg)
    return out
